# Initial kernel scaffold; baseline (speedup 1.0000x reference)
import jax
import jax.numpy as jnp
from jax.experimental import pallas as pl


def kernel(x, neighbor, W):
    raise NotImplementedError("write your pallas kernel here")


# speedup vs baseline: 1.5636x; 1.5636x over previous
import functools

import jax
import jax.numpy as jnp
from jax.experimental import pallas as pl


def _gcn_block(x_ref, nb_ref, w_ref, xo_ref, nbo_ref, *, bn, k, d_in):
    wt = w_ref[...].T
    nb = nb_ref[...]
    nb2 = nb.reshape(bn * k, d_in)
    out = jnp.dot(nb2, wt, preferred_element_type=jnp.float32)
    nbo_ref[...] = out.reshape(bn, k, -1)
    f = jnp.mean(nb, axis=1)
    xo_ref[...] = jnp.dot(x_ref[...] + f, wt,
                          preferred_element_type=jnp.float32)


@jax.jit
def kernel(x, neighbor, W):
    n, k, d_in = neighbor.shape
    d_out = W.shape[0]
    bn = 400
    grid = (n // bn,)
    body = functools.partial(_gcn_block, bn=bn, k=k, d_in=d_in)
    x_out, neighbor_out = pl.pallas_call(
        body,
        grid=grid,
        in_specs=[
            pl.BlockSpec((bn, d_in), lambda i: (i, 0)),
            pl.BlockSpec((bn, k, d_in), lambda i: (i, 0, 0)),
            pl.BlockSpec((d_out, d_in), lambda i: (0, 0)),
        ],
        out_specs=[
            pl.BlockSpec((bn, d_out), lambda i: (i, 0)),
            pl.BlockSpec((bn, k, d_out), lambda i: (i, 0, 0)),
        ],
        out_shape=[
            jax.ShapeDtypeStruct((n, d_out), jnp.float32),
            jax.ShapeDtypeStruct((n, k, d_out), jnp.float32),
        ],
    )(x, neighbor, W)
    return (x_out, neighbor_out)
